# baseline (speedup 1.0000x reference)
import functools

import jax
import jax.numpy as jnp
from jax import lax
from jax.experimental import pallas as pl
from jax.experimental.pallas import tpu as pltpu
from jax.experimental.pallas import tpu_sc as plsc

N = 16384
C = 1000
S = 8192

_INFO = plsc.get_sparse_core_info()
NC, NS, L = _INFO.num_cores, _INFO.num_subcores, _INFO.num_lanes
NW = NC * NS
CPW = (N - S) // NW
CB = 128
NBLK = CPW // CB
GROUPS = CB // L

_mesh = plsc.VectorSubcoreMesh(core_axis_name="c", subcore_axis_name="s")


@functools.partial(
    pl.kernel,
    out_type=jax.ShapeDtypeStruct((C, N), jnp.float32),
    mesh=_mesh,
    scratch_types=[
        pltpu.VMEM((CB,), jnp.int32),
        pltpu.VMEM((C, CB), jnp.float32),
        pltpu.SemaphoreType.DMA,
    ],
    compiler_params=pltpu.CompilerParams(needs_layout_passes=False),
)
def _one_hot_t_sc(x_hbm, zeros_hbm, out_hbm, idx_v, buf, sem):
    wid = lax.axis_index("s") * NC + lax.axis_index("c")
    wbase = S + wid * CPW

    pltpu.sync_copy(zeros_hbm, buf)

    lane = lax.iota(jnp.int32, L)
    ones = jnp.full((L,), 1.0, jnp.float32)
    zeros = jnp.zeros((L,), jnp.float32)

    d = None
    for c in range(NBLK):
        if d is not None:
            d.wait()
            for g in range(GROUPS):
                cols = lane + g * L
                cls = idx_v[pl.ds(g * L, L)]
                plsc.store_scatter(buf, [cls, cols], zeros)
        pltpu.sync_copy(x_hbm.at[pl.ds(wbase + c * CB, CB)], idx_v)
        for g in range(GROUPS):
            cols = lane + g * L
            cls = idx_v[pl.ds(g * L, L)]
            plsc.store_scatter(buf, [cls, cols], ones)
        d = pltpu.async_copy(
            buf, out_hbm.at[:, pl.ds(wbase + c * CB, CB)], sem
        )
    d.wait()


BC = 512


def _one_hot_t_tc_body(x_ref, buf_ref, out_ref):
    del buf_ref
    ids = x_ref[0]
    cls = lax.broadcasted_iota(jnp.int32, (C, BC), 0)
    out_ref[...] = (cls == ids).astype(jnp.float32)


_one_hot_t_tc = pl.pallas_call(
    _one_hot_t_tc_body,
    out_shape=jax.ShapeDtypeStruct((C, N), jnp.float32),
    grid=(S // BC,),
    in_specs=[
        pl.BlockSpec((1, 1, BC), lambda i: (i, 0, 0)),
        pl.BlockSpec(memory_space=pltpu.MemorySpace.HBM),
    ],
    out_specs=pl.BlockSpec((C, BC), lambda i: (0, i)),
    input_output_aliases={1: 0},
)


def kernel(x1):
    x = x1.astype(jnp.int32)
    zeros = jnp.zeros((C, CB), jnp.float32)
    partial = _one_hot_t_sc(x, zeros)
    x_tc = x[:S].reshape(S // BC, 1, BC)
    full = _one_hot_t_tc(x_tc, partial)
    return full.T


# speedup vs baseline: 1.0489x; 1.0489x over previous
import functools

import jax
import jax.numpy as jnp
from jax import lax
from jax.experimental import pallas as pl
from jax.experimental.pallas import tpu as pltpu
from jax.experimental.pallas import tpu_sc as plsc

N = 16384
C = 1000

_INFO = plsc.get_sparse_core_info()
NC, NS, L = _INFO.num_cores, _INFO.num_subcores, _INFO.num_lanes
NW = NC * NS
CPW = N // NW
CB = 128
NBLK = CPW // CB
GROUPS = CB // L

_mesh = plsc.VectorSubcoreMesh(core_axis_name="c", subcore_axis_name="s")


@functools.partial(
    pl.kernel,
    out_type=jax.ShapeDtypeStruct((C, N), jnp.float32),
    mesh=_mesh,
    scratch_types=[
        pltpu.VMEM((CPW,), jnp.int32),
        pltpu.VMEM((C, CB), jnp.float32),
        pltpu.SemaphoreType.DMA,
        pltpu.SemaphoreType.DMA,
    ],
    compiler_params=pltpu.CompilerParams(needs_layout_passes=False),
)
def _one_hot_t_sc(x_hbm, zeros_hbm, out_hbm, idx_v, buf, sem, zsem):
    wid = lax.axis_index("s") * NC + lax.axis_index("c")
    wbase = wid * CPW

    zd = pltpu.async_copy(zeros_hbm, buf, zsem)
    pltpu.sync_copy(x_hbm.at[pl.ds(wbase, CPW)], idx_v)
    zd.wait()

    lane = lax.iota(jnp.int32, L)
    ones = jnp.full((L,), 1.0, jnp.float32)
    zeros = jnp.zeros((L,), jnp.float32)

    d = None
    for c in range(NBLK):
        if d is not None:
            d.wait()
            for g in range(GROUPS):
                cols = lane + g * L
                cls = idx_v[pl.ds((c - 1) * CB + g * L, L)]
                plsc.store_scatter(buf, [cls, cols], zeros)
        for g in range(GROUPS):
            cols = lane + g * L
            cls = idx_v[pl.ds(c * CB + g * L, L)]
            plsc.store_scatter(buf, [cls, cols], ones)
        d = pltpu.async_copy(
            buf, out_hbm.at[:, pl.ds(wbase + c * CB, CB)], sem
        )
    d.wait()


def kernel(x1):
    x = x1.astype(jnp.int32)
    zeros = jnp.zeros((C, CB), jnp.float32)
    return _one_hot_t_sc(x, zeros).T


# speedup vs baseline: 1.0534x; 1.0042x over previous
import functools

import jax
import jax.numpy as jnp
from jax import lax
from jax.experimental import pallas as pl
from jax.experimental.pallas import tpu as pltpu
from jax.experimental.pallas import tpu_sc as plsc

N = 16384
C = 1000

_INFO = plsc.get_sparse_core_info()
NC, NS, L = _INFO.num_cores, _INFO.num_subcores, _INFO.num_lanes
NW = NC * NS
CPW = N // NW
CB = 128
NBLK = CPW // CB
GROUPS = CB // L

_mesh = plsc.VectorSubcoreMesh(core_axis_name="c", subcore_axis_name="s")


@functools.partial(
    pl.kernel,
    out_type=jax.ShapeDtypeStruct((C, N), jnp.float32),
    mesh=_mesh,
    scratch_types=[
        pltpu.VMEM((CPW,), jnp.int32),
        pltpu.VMEM((C, CB), jnp.float32),
        pltpu.SemaphoreType.DMA,
        pltpu.SemaphoreType.DMA,
    ],
    compiler_params=pltpu.CompilerParams(
        needs_layout_passes=False, skip_device_barrier=True
    ),
)
def _one_hot_t_sc(x_hbm, zeros_hbm, out_hbm, idx_v, buf, sem, zsem):
    wid = lax.axis_index("s") * NC + lax.axis_index("c")
    wbase = wid * CPW

    zd = pltpu.async_copy(zeros_hbm, buf, zsem)
    pltpu.sync_copy(x_hbm.at[pl.ds(wbase, CPW)], idx_v)
    zd.wait()

    lane = lax.iota(jnp.int32, L)
    ones = jnp.full((L,), 1.0, jnp.float32)
    zeros = jnp.zeros((L,), jnp.float32)

    d = None
    for c in range(NBLK):
        if d is not None:
            d.wait()
            for g in range(GROUPS):
                cols = lane + g * L
                cls = idx_v[pl.ds((c - 1) * CB + g * L, L)]
                plsc.store_scatter(buf, [cls, cols], zeros)
        for g in range(GROUPS):
            cols = lane + g * L
            cls = idx_v[pl.ds(c * CB + g * L, L)]
            plsc.store_scatter(buf, [cls, cols], ones)
        d = pltpu.async_copy(
            buf, out_hbm.at[:, pl.ds(wbase + c * CB, CB)], sem
        )
    d.wait()


def kernel(x1):
    x = x1.astype(jnp.int32)
    zeros = jnp.zeros((C, CB), jnp.float32)
    return _one_hot_t_sc(x, zeros).T
